# baseline (speedup 1.0000x reference)
import functools

import jax
import jax.numpy as jnp
from jax.experimental import pallas as pl


def _lrelu(v):
    return jnp.where(v > 0, v, 0.2 * v)


def _gat_kernel(x_ref, xT_ref, eT_ref,
                Ws1_ref, bs1_ref, Ws2_ref, bs2_ref,
                Wn1_ref, bn1_ref, Wn2_ref, bn2_ref,
                Wn1T_ref, bn1c_ref, Wn2T_ref, bn2c_ref,
                Wc1s_ref, bc1_ref, Wc1nT_ref,
                c06_ref, c04_ref, bc2_ref,
                out_ref):
    n = x_ref.shape[1]
    h = Ws2_ref.shape[1]
    dot = functools.partial(jnp.dot, preferred_element_type=jnp.float32)

    x = x_ref[0]
    xT = xT_ref[0]

    se = dot(_lrelu(dot(x, Ws1_ref[...]) + bs1_ref[...]), Ws2_ref[...]) + bs2_ref[...]
    ne = dot(_lrelu(dot(x, Wn1_ref[...]) + bn1_ref[...]), Wn2_ref[...]) + bn2_ref[...]
    neT = dot(Wn2T_ref[...], _lrelu(dot(Wn1T_ref[...], xT) + bn1c_ref[...])) + bn2c_ref[...]

    ps = dot(se, Wc1s_ref[...]) + bc1_ref[...]
    pnT = dot(Wc1nT_ref[...], neT)

    u06 = dot(ps, c06_ref[...])
    v06 = dot(c06_ref[...].T, pnT)
    c04 = c04_ref[...]

    def body(k, acc):
        oh_col = (jax.lax.broadcasted_iota(jnp.int32, (h, 1), 0) == k).astype(jnp.float32)
        oh_row = (jax.lax.broadcasted_iota(jnp.int32, (1, h), 1) == k).astype(jnp.float32)
        ps_k = dot(ps, oh_col)
        pn_k = dot(oh_row, pnT)
        c_k = dot(oh_row, c04)
        return acc + c_k * jnp.abs(ps_k + pn_k)

    acc = jax.lax.fori_loop(0, h, body, jnp.zeros((n, n), jnp.float32))
    imp = u06 + v06 + acc + bc2_ref[...]

    ii = jax.lax.broadcasted_iota(jnp.int32, (n, n), 0)
    jj = jax.lax.broadcasted_iota(jnp.int32, (n, n), 1)
    mask = (eT_ref[0] != 0) & (ii != jj)
    logits = jnp.where(mask, imp, -1e30)
    m = jnp.max(logits, axis=1, keepdims=True)
    e = jnp.exp(logits - m)
    s = jnp.sum(e, axis=1, keepdims=True)
    w = (e / s) * mask.astype(jnp.float32)

    sum_nb = dot(w, ne)
    has = jnp.max(mask.astype(jnp.float32), axis=1, keepdims=True) > 0
    out_ref[0] = jnp.where(has, sum_nb + se, 0.0)


def kernel(nodes, edges, Ws1, bs1, Ws2, bs2, Wn1, bn1, Wn2, bn2, Wc1, bc1, Wc2, bc2):
    b, n = nodes.shape[0], nodes.shape[1]
    d = nodes.shape[2] * nodes.shape[3]
    h = Ws2.shape[1]

    x = nodes.reshape(b, n, d)
    xT = jnp.swapaxes(x, 1, 2)
    eT = jnp.swapaxes(edges, 1, 2)

    row = lambda v: v.reshape(1, -1)
    col = lambda v: v.reshape(-1, 1)

    operands = (
        x, xT, eT,
        Ws1, row(bs1), Ws2, row(bs2),
        Wn1, row(bn1), Wn2, row(bn2),
        Wn1.T, col(bn1), Wn2.T, col(bn2),
        Wc1[:h], row(bc1), Wc1[h:].T,
        0.6 * Wc2, 0.4 * Wc2, bc2.reshape(1, 1),
    )

    def bspec(a):
        if a.ndim == 3:
            return pl.BlockSpec((1,) + a.shape[1:], lambda i: (i, 0, 0))
        return pl.BlockSpec(a.shape, lambda i: (0,) * a.ndim)

    return pl.pallas_call(
        _gat_kernel,
        grid=(b,),
        in_specs=[bspec(a) for a in operands],
        out_specs=pl.BlockSpec((1, n, h), lambda i: (i, 0, 0)),
        out_shape=jax.ShapeDtypeStruct((b, n, h), jnp.float32),
    )(*operands)


# speedup vs baseline: 2.1795x; 2.1795x over previous
import functools

import jax
import jax.numpy as jnp
from jax.experimental import pallas as pl


def _lrelu(v):
    return jnp.where(v > 0, v, 0.2 * v)


def _gat_kernel(x_ref, xT_ref, eT_ref,
                Ws1_ref, bs1_ref, Ws2_ref, bs2_ref,
                Wn1_ref, bn1_ref, Wn2_ref, bn2_ref,
                Wn1T_ref, bn1c_ref, Wn2T_ref, bn2c_ref,
                Wc1s_ref, bc1_ref, Wc1nT_ref,
                c06_ref, c04_ref, bc2_ref,
                out_ref):
    n = x_ref.shape[1]
    h = Ws2_ref.shape[1]
    dot = functools.partial(jnp.dot, preferred_element_type=jnp.float32)

    x = x_ref[0]
    xT = xT_ref[0]

    se = dot(_lrelu(dot(x, Ws1_ref[...]) + bs1_ref[...]), Ws2_ref[...]) + bs2_ref[...]
    ne = dot(_lrelu(dot(x, Wn1_ref[...]) + bn1_ref[...]), Wn2_ref[...]) + bn2_ref[...]
    neT = dot(Wn2T_ref[...], _lrelu(dot(Wn1T_ref[...], xT) + bn1c_ref[...])) + bn2c_ref[...]

    ps = dot(se, Wc1s_ref[...]) + bc1_ref[...]
    pnT = dot(Wc1nT_ref[...], neT)

    u06 = dot(ps, c06_ref[...])
    v06 = dot(c06_ref[...].T, pnT)
    c04 = c04_ref[...]

    acc = jnp.zeros((n, n), jnp.float32)
    for k in range(h):
        ps_k = jax.lax.slice(ps, (0, k), (n, k + 1))
        pn_k = jax.lax.slice(pnT, (k, 0), (k + 1, n))
        c_k = jax.lax.slice(c04, (k, 0), (k + 1, 1))
        acc = acc + c_k * jnp.abs(ps_k + pn_k)
    imp = u06 + v06 + acc + bc2_ref[...]

    ii = jax.lax.broadcasted_iota(jnp.int32, (n, n), 0)
    jj = jax.lax.broadcasted_iota(jnp.int32, (n, n), 1)
    mask = (eT_ref[0] != 0) & (ii != jj)
    logits = jnp.where(mask, imp, -1e30)
    m = jnp.max(logits, axis=1, keepdims=True)
    e = jnp.exp(logits - m)
    s = jnp.sum(e, axis=1, keepdims=True)
    w = (e / s) * mask.astype(jnp.float32)

    sum_nb = dot(w, ne)
    has = jnp.max(mask.astype(jnp.float32), axis=1, keepdims=True) > 0
    out_ref[0] = jnp.where(has, sum_nb + se, 0.0)


def kernel(nodes, edges, Ws1, bs1, Ws2, bs2, Wn1, bn1, Wn2, bn2, Wc1, bc1, Wc2, bc2):
    b, n = nodes.shape[0], nodes.shape[1]
    d = nodes.shape[2] * nodes.shape[3]
    h = Ws2.shape[1]

    x = nodes.reshape(b, n, d)
    xT = jnp.swapaxes(x, 1, 2)
    eT = jnp.swapaxes(edges, 1, 2)

    row = lambda v: v.reshape(1, -1)
    col = lambda v: v.reshape(-1, 1)

    operands = (
        x, xT, eT,
        Ws1, row(bs1), Ws2, row(bs2),
        Wn1, row(bn1), Wn2, row(bn2),
        Wn1.T, col(bn1), Wn2.T, col(bn2),
        Wc1[:h], row(bc1), Wc1[h:].T,
        0.6 * Wc2, 0.4 * Wc2, bc2.reshape(1, 1),
    )

    def bspec(a):
        if a.ndim == 3:
            return pl.BlockSpec((1,) + a.shape[1:], lambda i: (i, 0, 0))
        return pl.BlockSpec(a.shape, lambda i: (0,) * a.ndim)

    return pl.pallas_call(
        _gat_kernel,
        grid=(b,),
        in_specs=[bspec(a) for a in operands],
        out_specs=pl.BlockSpec((1, n, h), lambda i: (i, 0, 0)),
        out_shape=jax.ShapeDtypeStruct((b, n, h), jnp.float32),
    )(*operands)
